# baseline (device time: 101714 ns/iter reference)
import jax
import jax.numpy as jnp
from jax import lax
from jax.experimental import pallas as pl
from jax.experimental.pallas import tpu as pltpu

N_DEV = 4
N_SUB = 4
CW, CCW = 0, 1


def _gelu(z):
    return 0.5 * z * (1.0 + jnp.tanh(0.7978845608 * (z + 0.044715 * z * z * z)))


def kernel(A, B):
    m, k = A.shape
    _, n = B.shape
    half = m // 2
    mc = half // N_DEV
    sub = mc // N_SUB

    def body(a_ref, b_ref, out_ref, acc_ref, b16_ref, comm_cw, comm_ccw,
             send_sems, recv_sems):
        my = lax.axis_index("i")
        left = (my - 1) % N_DEV
        right = (my + 1) % N_DEV

        barrier_sem = pltpu.get_barrier_semaphore()
        for nbr in (left, right):
            pl.semaphore_signal(
                barrier_sem, inc=1,
                device_id=(nbr,), device_id_type=pl.DeviceIdType.MESH,
            )
        pl.semaphore_wait(barrier_sem, 2)

        b16_ref[:, :] = b_ref[:, :].astype(jnp.bfloat16)

        def top_rows(c):
            return pl.ds((c % N_DEV) * mc, mc)

        def bot_rows(c):
            return pl.ds(half + (c % N_DEV) * mc, mc)

        def top_sub(c, s):
            return pl.ds((c % N_DEV) * mc + s * sub, sub)

        def bot_sub(c, s):
            return pl.ds(half + (c % N_DEV) * mc + s * sub, sub)

        def compute_chunk(rows):
            acc_ref[rows, :] = jnp.dot(
                a_ref[rows, :].astype(jnp.bfloat16), b16_ref[:, :],
                preferred_element_type=jnp.float32,
            ).astype(jnp.bfloat16)

        def rs_rdma(h, s):
            slot = h % 2
            cw = pltpu.make_async_remote_copy(
                src_ref=acc_ref.at[top_sub(my - h, s), :],
                dst_ref=comm_cw.at[s, slot],
                send_sem=send_sems.at[CW, s, h],
                recv_sem=recv_sems.at[CW, s, h],
                device_id=(right,),
                device_id_type=pl.DeviceIdType.MESH,
            )
            ccw = pltpu.make_async_remote_copy(
                src_ref=acc_ref.at[bot_sub(my + h, s), :],
                dst_ref=comm_ccw.at[s, slot],
                send_sem=send_sems.at[CCW, s, h],
                recv_sem=recv_sems.at[CCW, s, h],
                device_id=(left,),
                device_id_type=pl.DeviceIdType.MESH,
            )
            return cw, ccw

        def ag_rdma(h, s):
            cw = pltpu.make_async_remote_copy(
                src_ref=acc_ref.at[top_sub(my + 1 - h, s), :],
                dst_ref=acc_ref.at[top_sub(my + 1 - h, s), :],
                send_sem=send_sems.at[CW, s, N_DEV - 1 + h],
                recv_sem=recv_sems.at[CW, s, N_DEV - 1 + h],
                device_id=(right,),
                device_id_type=pl.DeviceIdType.MESH,
            )
            ccw = pltpu.make_async_remote_copy(
                src_ref=acc_ref.at[bot_sub(my - 1 + h, s), :],
                dst_ref=acc_ref.at[bot_sub(my - 1 + h, s), :],
                send_sem=send_sems.at[CCW, s, N_DEV - 1 + h],
                recv_sem=recv_sems.at[CCW, s, N_DEV - 1 + h],
                device_id=(left,),
                device_id_type=pl.DeviceIdType.MESH,
            )
            return cw, ccw

        rs_hops = {}
        ag_hops = {}
        for s in range(N_SUB):
            compute_chunk(top_sub(my, s))
            rs_hops[(0, s)] = rs_rdma(0, s)
            rs_hops[(0, s)][0].start()
        for s in range(N_SUB):
            compute_chunk(bot_sub(my, s))
            rs_hops[(0, s)][1].start()

        for h in range(N_DEV - 2):
            compute_chunk(top_rows(my - h - 1))
            compute_chunk(bot_rows(my + h + 1))
            slot = h % 2
            for s in range(N_SUB):
                cw, ccw = rs_hops[(h, s)]
                rs_hops[(h + 1, s)] = rs_rdma(h + 1, s)
                cw.wait_recv()
                rt = top_sub(my - h - 1, s)
                acc_ref[rt, :] = acc_ref[rt, :] + comm_cw[s, slot, :, :]
                rs_hops[(h + 1, s)][0].start()
                ccw.wait_recv()
                rb = bot_sub(my + h + 1, s)
                acc_ref[rb, :] = acc_ref[rb, :] + comm_ccw[s, slot, :, :]
                rs_hops[(h + 1, s)][1].start()

        hl = N_DEV - 2
        compute_chunk(top_rows(my + 1))
        compute_chunk(bot_rows(my - 1))
        slot = hl % 2
        for s in range(N_SUB):
            cw, ccw = rs_hops[(hl, s)]
            ag_hops[(0, s)] = ag_rdma(0, s)
            cw.wait_recv()
            rt = top_sub(my + 1, s)
            g = _gelu(
                acc_ref[rt, :].astype(jnp.float32)
                + comm_cw[s, slot, :, :].astype(jnp.float32)
            )
            acc_ref[rt, :] = g.astype(jnp.bfloat16)
            ag_hops[(0, s)][0].start()
            out_ref[rt, :] = g
            ccw.wait_recv()
            rb = bot_sub(my - 1, s)
            g = _gelu(
                acc_ref[rb, :].astype(jnp.float32)
                + comm_ccw[s, slot, :, :].astype(jnp.float32)
            )
            acc_ref[rb, :] = g.astype(jnp.bfloat16)
            ag_hops[(0, s)][1].start()
            out_ref[rb, :] = g

        for h in range(N_DEV - 1):
            last = h == N_DEV - 2
            for s in range(N_SUB):
                cw, ccw = ag_hops[(h, s)]
                if not last:
                    ag_hops[(h + 1, s)] = ag_rdma(h + 1, s)
                cw.wait_recv()
                if not last:
                    ag_hops[(h + 1, s)][0].start()
                rt = top_sub(my - h, s)
                out_ref[rt, :] = acc_ref[rt, :].astype(jnp.float32)
                ccw.wait_recv()
                if not last:
                    ag_hops[(h + 1, s)][1].start()
                rb = bot_sub(my + h, s)
                out_ref[rb, :] = acc_ref[rb, :].astype(jnp.float32)

        for h in range(N_DEV - 1):
            for s in range(N_SUB):
                for d in rs_rdma(h, s) + ag_rdma(h, s):
                    d.wait_send()

    return pl.pallas_call(
        body,
        out_shape=jax.ShapeDtypeStruct((m, n), jnp.float32),
        in_specs=[
            pl.BlockSpec(memory_space=pltpu.VMEM),
            pl.BlockSpec(memory_space=pltpu.VMEM),
        ],
        out_specs=pl.BlockSpec(memory_space=pltpu.VMEM),
        scratch_shapes=[
            pltpu.VMEM((m, n), jnp.bfloat16),
            pltpu.VMEM((k, n), jnp.bfloat16),
            pltpu.VMEM((N_SUB, 2, sub, n), jnp.bfloat16),
            pltpu.VMEM((N_SUB, 2, sub, n), jnp.bfloat16),
            pltpu.SemaphoreType.DMA((2, N_SUB, 2 * (N_DEV - 1))),
            pltpu.SemaphoreType.DMA((2, N_SUB, 2 * (N_DEV - 1))),
        ],
        compiler_params=pltpu.CompilerParams(
            collective_id=0, vmem_limit_bytes=100 * 1024 * 1024
        ),
    )(A, B)


# device time: 94622 ns/iter; 1.0750x vs baseline; 1.0750x over previous
import jax
import jax.numpy as jnp
from jax import lax
from jax.experimental import pallas as pl
from jax.experimental.pallas import tpu as pltpu

N_DEV = 4
N_SUB = 2
CW, CCW = 0, 1


def _gelu(z):
    return 0.5 * z * (1.0 + jnp.tanh(0.7978845608 * (z + 0.044715 * z * z * z)))


def kernel(A, B):
    m, k = A.shape
    _, n = B.shape
    half = m // 2
    mc = half // N_DEV
    sub = mc // N_SUB

    def body(a_ref, b_ref, out_ref, acc_ref, b16_ref, comm_cw, comm_ccw,
             stage_ref, send_sems, recv_sems, copy_sems):
        my = lax.axis_index("i")
        left = (my - 1) % N_DEV
        right = (my + 1) % N_DEV

        barrier_sem = pltpu.get_barrier_semaphore()
        for nbr in (left, right):
            pl.semaphore_signal(
                barrier_sem, inc=1,
                device_id=(nbr,), device_id_type=pl.DeviceIdType.MESH,
            )
        pl.semaphore_wait(barrier_sem, 2)

        b16_ref[:, :] = b_ref[:, :].astype(jnp.bfloat16)

        n_slots = stage_ref.shape[0]
        flush_state = {"i": 0, "pending": {}}

        def flush(rows, value):
            slot = flush_state["i"] % n_slots
            prev = flush_state["pending"].pop(slot, None)
            if prev is not None:
                prev.wait()
            stage_ref[slot, :, :] = value
            cp = pltpu.make_async_copy(
                stage_ref.at[slot], out_ref.at[rows, :], copy_sems.at[slot]
            )
            cp.start()
            flush_state["pending"][slot] = cp
            flush_state["i"] += 1

        def top_rows(c):
            return pl.ds((c % N_DEV) * mc, mc)

        def bot_rows(c):
            return pl.ds(half + (c % N_DEV) * mc, mc)

        def top_sub(c, s):
            return pl.ds((c % N_DEV) * mc + s * sub, sub)

        def bot_sub(c, s):
            return pl.ds(half + (c % N_DEV) * mc + s * sub, sub)

        def compute_chunk(rows):
            acc_ref[rows, :] = jnp.dot(
                a_ref[rows, :].astype(jnp.bfloat16), b16_ref[:, :],
                preferred_element_type=jnp.float32,
            ).astype(jnp.bfloat16)

        def rs_rdma(h, s):
            slot = h % 2
            cw = pltpu.make_async_remote_copy(
                src_ref=acc_ref.at[top_sub(my - h, s), :],
                dst_ref=comm_cw.at[s, slot],
                send_sem=send_sems.at[CW, s, h],
                recv_sem=recv_sems.at[CW, s, h],
                device_id=(right,),
                device_id_type=pl.DeviceIdType.MESH,
            )
            ccw = pltpu.make_async_remote_copy(
                src_ref=acc_ref.at[bot_sub(my + h, s), :],
                dst_ref=comm_ccw.at[s, slot],
                send_sem=send_sems.at[CCW, s, h],
                recv_sem=recv_sems.at[CCW, s, h],
                device_id=(left,),
                device_id_type=pl.DeviceIdType.MESH,
            )
            return cw, ccw

        def ag_rdma(h, s):
            cw = pltpu.make_async_remote_copy(
                src_ref=acc_ref.at[top_sub(my + 1 - h, s), :],
                dst_ref=acc_ref.at[top_sub(my + 1 - h, s), :],
                send_sem=send_sems.at[CW, s, N_DEV - 1 + h],
                recv_sem=recv_sems.at[CW, s, N_DEV - 1 + h],
                device_id=(right,),
                device_id_type=pl.DeviceIdType.MESH,
            )
            ccw = pltpu.make_async_remote_copy(
                src_ref=acc_ref.at[bot_sub(my - 1 + h, s), :],
                dst_ref=acc_ref.at[bot_sub(my - 1 + h, s), :],
                send_sem=send_sems.at[CCW, s, N_DEV - 1 + h],
                recv_sem=recv_sems.at[CCW, s, N_DEV - 1 + h],
                device_id=(left,),
                device_id_type=pl.DeviceIdType.MESH,
            )
            return cw, ccw

        rs_hops = {}
        ag_hops = {}
        for s in range(N_SUB):
            compute_chunk(top_sub(my, s))
            rs_hops[(0, s)] = rs_rdma(0, s)
            rs_hops[(0, s)][0].start()
        for s in range(N_SUB):
            compute_chunk(bot_sub(my, s))
            rs_hops[(0, s)][1].start()

        for h in range(N_DEV - 2):
            compute_chunk(top_rows(my - h - 1))
            compute_chunk(bot_rows(my + h + 1))
            slot = h % 2
            for s in range(N_SUB):
                cw, ccw = rs_hops[(h, s)]
                rs_hops[(h + 1, s)] = rs_rdma(h + 1, s)
                cw.wait_recv()
                rt = top_sub(my - h - 1, s)
                acc_ref[rt, :] = acc_ref[rt, :] + comm_cw[s, slot, :, :]
                rs_hops[(h + 1, s)][0].start()
                ccw.wait_recv()
                rb = bot_sub(my + h + 1, s)
                acc_ref[rb, :] = acc_ref[rb, :] + comm_ccw[s, slot, :, :]
                rs_hops[(h + 1, s)][1].start()

        hl = N_DEV - 2
        compute_chunk(top_rows(my + 1))
        compute_chunk(bot_rows(my - 1))
        slot = hl % 2
        for s in range(N_SUB):
            cw, ccw = rs_hops[(hl, s)]
            ag_hops[(0, s)] = ag_rdma(0, s)
            cw.wait_recv()
            rt = top_sub(my + 1, s)
            g = _gelu(
                acc_ref[rt, :].astype(jnp.float32)
                + comm_cw[s, slot, :, :].astype(jnp.float32)
            )
            acc_ref[rt, :] = g.astype(jnp.bfloat16)
            ag_hops[(0, s)][0].start()
            flush(rt, g)
            ccw.wait_recv()
            rb = bot_sub(my - 1, s)
            g = _gelu(
                acc_ref[rb, :].astype(jnp.float32)
                + comm_ccw[s, slot, :, :].astype(jnp.float32)
            )
            acc_ref[rb, :] = g.astype(jnp.bfloat16)
            ag_hops[(0, s)][1].start()
            flush(rb, g)

        for h in range(N_DEV - 1):
            last = h == N_DEV - 2
            for s in range(N_SUB):
                cw, ccw = ag_hops[(h, s)]
                if not last:
                    ag_hops[(h + 1, s)] = ag_rdma(h + 1, s)
                cw.wait_recv()
                if not last:
                    ag_hops[(h + 1, s)][0].start()
                rt = top_sub(my - h, s)
                flush(rt, acc_ref[rt, :].astype(jnp.float32))
                ccw.wait_recv()
                if not last:
                    ag_hops[(h + 1, s)][1].start()
                rb = bot_sub(my + h, s)
                flush(rb, acc_ref[rb, :].astype(jnp.float32))

        for h in range(N_DEV - 1):
            for s in range(N_SUB):
                for d in rs_rdma(h, s) + ag_rdma(h, s):
                    d.wait_send()
        for cp in flush_state["pending"].values():
            cp.wait()

    return pl.pallas_call(
        body,
        out_shape=jax.ShapeDtypeStruct((m, n), jnp.float32),
        in_specs=[
            pl.BlockSpec(memory_space=pltpu.VMEM),
            pl.BlockSpec(memory_space=pltpu.VMEM),
        ],
        out_specs=pl.BlockSpec(memory_space=pl.ANY),
        scratch_shapes=[
            pltpu.VMEM((m, n), jnp.bfloat16),
            pltpu.VMEM((k, n), jnp.bfloat16),
            pltpu.VMEM((N_SUB, 2, sub, n), jnp.bfloat16),
            pltpu.VMEM((N_SUB, 2, sub, n), jnp.bfloat16),
            pltpu.VMEM((4, sub, n), jnp.float32),
            pltpu.SemaphoreType.DMA((2, N_SUB, 2 * (N_DEV - 1))),
            pltpu.SemaphoreType.DMA((2, N_SUB, 2 * (N_DEV - 1))),
            pltpu.SemaphoreType.DMA((4,)),
        ],
        compiler_params=pltpu.CompilerParams(
            collective_id=0, vmem_limit_bytes=100 * 1024 * 1024
        ),
    )(A, B)
